# baseline (speedup 1.0000x reference)
import functools

import jax
import jax.numpy as jnp
import numpy as np
from jax import lax
from jax.experimental import pallas as pl
from jax.experimental.pallas import tpu as pltpu
from jax.experimental.pallas import tpu_sc as plsc

_ANCHORS = [[42.31, 55.41], [102.17, 128.3], [161.79, 259.17], [303.08, 154.9], [359.56, 320.23]]
_ABW = tuple(float(np.float32(np.float32(a[0]) / np.float32(512.0 / 14.0) / np.float32(14.0))) for a in _ANCHORS)
_ABH = tuple(float(np.float32(np.float32(a[1]) / np.float32(512.0 / 14.0) / np.float32(14.0))) for a in _ANCHORS)
_NEG = float(np.float32(-3e38))
_LN2 = 0.6931471805599453
_BS, _A, _N, _NP = 64, 5, 196, 208
_Q = 1024
_CHUNKS = _NP // 16


def _ln(x):
    bits = lax.bitcast_convert_type(x, jnp.int32)
    e = ((bits >> 23) & 0xFF) - 127
    m = lax.bitcast_convert_type((bits & 0x007FFFFF) | 0x3F800000, jnp.float32)
    s = (m - 1.0) / (m + 1.0)
    s2 = s * s
    p = 1.0 + s2 * (1.0 / 3.0 + s2 * (0.2 + s2 * (1.0 / 7.0 + s2 * (1.0 / 9.0))))
    ln = e.astype(jnp.float32) * _LN2 + 2.0 * s * p
    return jnp.where(x > 0.0, ln, -jnp.inf)


def _sig(x):
    return 1.0 / (1.0 + jnp.exp(-x))


def _sc_body(imgs_per, nc, pred_h, lab_h, sg_h, gts_h, pred_v, lab_v, gt_v, sg_v):
    wid = lax.axis_index("s") * nc + lax.axis_index("c")
    lane = lax.iota(jnp.int32, 16)
    zeros16 = jnp.zeros((16,), jnp.float32)

    def one_image(i, carry):
        b = wid * imgs_per + i
        pltpu.sync_copy(pred_h.at[b], pred_v)
        pltpu.sync_copy(lab_h.at[b], lab_v)

        def zbody(k, c):
            sl = pl.ds(k * 16, 16)
            for r in range(6):
                sg_v[r, sl] = zeros16
            return c

        lax.fori_loop(0, _Q // 16, zbody, jnp.int32(0))

        def abody(k, acc):
            acc_t, anyv = acc
            sl = pl.ds(k * 16, 16)
            n_i = lane + k * 16
            colf = lax.rem(n_i, 14).astype(jnp.float32)
            rowf = lax.div(n_i, 14).astype(jnp.float32)
            l0 = lab_v[0, sl]
            l3 = lab_v[3, sl]
            l4 = lab_v[4, sl]
            tw = lab_v[5, sl]
            th = lab_v[6, sl]
            valid = l0 != 0.0
            vldf = jnp.where(valid, 1.0, 0.0).astype(jnp.float32)
            txf = l3 + colf
            tyf = l4 + rowf
            wi = txf.astype(jnp.int32)
            hj = tyf.astype(jnp.int32)
            gtx = txf / 14.0
            gty = tyf / 14.0
            gx1 = gtx - 0.5 * tw
            gx2 = gtx + 0.5 * tw
            gy1 = gty - 0.5 * th
            gy2 = gty + 0.5 * th
            ga = tw * th
            gc = jnp.where(valid, -0.6 * ga, _NEG)
            gt_v[0, sl] = gx1
            gt_v[1, sl] = gx2
            gt_v[2, sl] = gy1
            gt_v[3, sl] = gy2
            gt_v[4, sl] = gc
            best = jnp.full((16,), -1.0, jnp.float32)
            bid = jnp.zeros((16,), jnp.int32)
            aws = jnp.full((16,), _ABW[0], jnp.float32)
            ahs = jnp.full((16,), _ABH[0], jnp.float32)
            for a in range(_A):
                inter = jnp.minimum(tw, _ABW[a]) * jnp.minimum(th, _ABH[a])
                u = ga + (_ABW[a] * _ABH[a]) - inter
                iou = jnp.maximum(inter / u, 0.0)
                cnd = iou > best
                best = jnp.where(cnd, iou, best)
                bid = jnp.where(cnd, a, bid)
                aws = jnp.where(cnd, _ABW[a], aws)
                ahs = jnp.where(cnd, _ABH[a], ahs)
            wg = jnp.minimum(wi, 13)
            hg = jnp.minimum(hj, 13)
            cg = hg * 14 + wg
            ch0 = bid * 7
            p0g = plsc.load_gather(pred_v, [ch0, cg])
            p1g = plsc.load_gather(pred_v, [ch0 + 1, cg])
            p2g = plsc.load_gather(pred_v, [ch0 + 2, cg])
            p3g = plsc.load_gather(pred_v, [ch0 + 3, cg])
            p4g = plsc.load_gather(pred_v, [ch0 + 4, cg])
            p5g = plsc.load_gather(pred_v, [ch0 + 5, cg])
            p6g = plsc.load_gather(pred_v, [ch0 + 6, cg])
            row1 = jnp.full((16,), 1, jnp.int32)
            cv1 = plsc.load_gather(lab_v, [row1, cg])
            cv2 = plsc.load_gather(lab_v, [row1 + 1, cg])
            ax = _sig(p3g)
            ay = _sig(p4g)
            lnw = _ln(tw / aws)
            lnh = _ln(th / ahs)
            truth_x = txf - wi.astype(jnp.float32)
            truth_y = tyf - hj.astype(jnp.float32)
            scale = 2.0 - truth_x * truth_y
            px = (ax + wg.astype(jnp.float32)) / 14.0
            py = (ay + hg.astype(jnp.float32)) / 14.0
            pw = jnp.exp(p5g) * aws
            ph = jnp.exp(p6g) * ahs
            px1 = px - 0.5 * pw
            px2 = px + 0.5 * pw
            py1 = py - 0.5 * ph
            py2 = py + 0.5 * ph
            xi1 = jnp.maximum(px1, gx1)
            xi2 = jnp.minimum(px2, gx2)
            yi1 = jnp.maximum(py1, gy1)
            yi2 = jnp.minimum(py2, gy2)
            inter = jnp.maximum(xi2 - xi1, 0.0) * jnp.maximum(yi2 - yi1, 0.0)
            u = pw * ph + ga - inter
            iou_t = jnp.maximum(inter / u, 0.0)
            obj5 = 5.0 * (p0g - iou_t)
            c1 = scale * (ax - truth_x)
            c2 = scale * (ay - truth_y)
            c3 = scale * (p5g - lnw)
            c4 = scale * (p6g - lnh)
            cls = (p1g - cv1) * (p1g - cv1) + (p2g - cv2) * (p2g - cv2)
            inb = valid & (wi <= 13) & (hj <= 13)
            q = bid * _N + hg * 14 + wg
            onesf = jnp.full((16,), 1.0, jnp.float32)
            for ci, cc in enumerate((c1, c2, c3, c4, obj5, onesf)):
                plsc.addupdate_scatter(sg_v, [jnp.full((16,), ci, jnp.int32), q],
                                       cc, mask=inb)
            acc_t = acc_t + jnp.where(valid, cls, 0.0)
            anyv = jnp.maximum(anyv, vldf)
            return acc_t, anyv

        acc_t, anyv = lax.fori_loop(0, _CHUNKS, abody, (zeros16, zeros16))
        cls_total = jnp.sum(acc_t)
        anym = jnp.max(anyv)
        gt_v[5, pl.ds(0, 16)] = jnp.where(
            lane == 0, cls_total, jnp.where(lane == 1, anym, 0.0))
        pltpu.sync_copy(sg_v, sg_h.at[b])
        pltpu.sync_copy(gt_v, gts_h.at[b])
        return carry

    lax.fori_loop(0, imgs_per, one_image, jnp.int32(0))


def _tc_body(predT_ref, sg_ref, gts_ref, aux_ref, coef_ref, out_ref):
    coefs = coef_ref[0, 0]
    abw = aux_ref[0].reshape(8, 128)
    abh = aux_ref[1].reshape(8, 128)
    colf = aux_ref[2].reshape(8, 128)
    rowf = aux_ref[3].reshape(8, 128)
    p0 = predT_ref[0, 0].reshape(8, 128)
    p3 = predT_ref[0, 3].reshape(8, 128)
    p4 = predT_ref[0, 4].reshape(8, 128)
    p5 = predT_ref[0, 5].reshape(8, 128)
    p6 = predT_ref[0, 6].reshape(8, 128)
    ax = _sig(p3)
    ay = _sig(p4)
    px = (ax + colf) / 14.0
    py = (ay + rowf) / 14.0
    pw = jnp.exp(p5) * abw
    ph = jnp.exp(p6) * abh
    px1 = px - 0.5 * pw
    px2 = px + 0.5 * pw
    py1 = py - 0.5 * ph
    py2 = py + 0.5 * ph
    pam = -0.6 * (pw * ph)

    def gbody(g, m):
        gx1s = gts_ref[0, 0, g]
        gx2s = gts_ref[0, 1, g]
        gy1s = gts_ref[0, 2, g]
        gy2s = gts_ref[0, 3, g]
        gcs = gts_ref[0, 4, g]
        xi1 = jnp.maximum(px1, gx1s)
        xi2 = jnp.minimum(px2, gx2s)
        yi1 = jnp.maximum(py1, gy1s)
        yi2 = jnp.minimum(py2, gy2s)
        inter = jnp.maximum(xi2 - xi1, 0.0) * jnp.maximum(yi2 - yi1, 0.0)
        return jnp.maximum(m, 1.6 * inter + gcs)

    m = lax.fori_loop(0, _N, gbody, jnp.full((8, 128), _NEG, jnp.float32))
    noobj = (m + pam) <= 0.0
    ol = jnp.where(noobj, 0.5 * p0, 0.0)
    mc = sg_ref[0, 5].reshape(8, 128)
    omc = 1.0 - mc
    acc = jnp.zeros((8, 128), jnp.float32)
    for ci, co in enumerate((coefs * (ax - 0.5), coefs * (ay - 0.5),
                             coefs * p5, coefs * p6, ol)):
        tg = sg_ref[0, ci].reshape(8, 128)
        bc = mc * tg + omc * co
        acc = acc + bc * bc
    cls_s = gts_ref[0, 5, 0]
    anyv_s = gts_ref[0, 5, 1]
    pb = jnp.where(anyv_s > 0.0, jnp.sum(acc) + cls_s, 0.0)
    out_ref[0, 0, 0] = pb


def kernel(pred, label, seen):
    try:
        info = plsc.get_sparse_core_info()
        nc, ns = info.num_cores, info.num_subcores
    except Exception:
        nc, ns = 2, 16
    nw = nc * ns
    imgs_per = _BS // nw
    pred2 = jnp.pad(pred.reshape(_BS, 35, _N), ((0, 0), (0, 0), (0, _NP - _N)))
    lab2 = jnp.pad(label.reshape(_BS, 7, _N), ((0, 0), (0, 0), (0, _NP - _N)))
    predT = jnp.pad(
        pred.reshape(_BS, _A, 7, _N).transpose(0, 2, 1, 3).reshape(_BS, 7, _A * _N),
        ((0, 0), (0, 0), (0, _Q - _A * _N)))
    coef = jnp.where(jnp.asarray(seen) < 12800, jnp.float32(0.01),
                     jnp.float32(0.0)).reshape(1, 1)
    q = np.arange(_Q)
    a_of_q = np.minimum(q // _N, _A - 1)
    n_of_q = q % _N
    aux = np.zeros((6, _Q), np.float32)
    aux[0] = np.where(q < _A * _N, np.asarray(_ABW, np.float32)[a_of_q], 1.0)
    aux[1] = np.where(q < _A * _N, np.asarray(_ABH, np.float32)[a_of_q], 1.0)
    aux[2] = np.where(q < _A * _N, (n_of_q % 14).astype(np.float32), 0.0)
    aux[3] = np.where(q < _A * _N, (n_of_q // 14).astype(np.float32), 0.0)
    aux = jnp.asarray(aux)

    mesh = plsc.VectorSubcoreMesh(core_axis_name="c", subcore_axis_name="s",
                                  num_cores=nc, num_subcores=ns)
    sg, gts = pl.kernel(
        functools.partial(_sc_body, imgs_per, nc),
        out_type=(jax.ShapeDtypeStruct((_BS, 6, _Q), jnp.float32),
                  jax.ShapeDtypeStruct((_BS, 6, _NP), jnp.float32)),
        mesh=mesh,
        compiler_params=pltpu.CompilerParams(use_tc_tiling_on_sc=False,
                                             needs_layout_passes=False),
        scratch_types=[
            pltpu.VMEM((35, _NP), jnp.float32),
            pltpu.VMEM((7, _NP), jnp.float32),
            pltpu.VMEM((6, _NP), jnp.float32),
            pltpu.VMEM((6, _Q), jnp.float32),
        ],
    )(pred2, lab2)

    out = pl.pallas_call(
        _tc_body,
        grid=(_BS,),
        in_specs=[
            pl.BlockSpec((1, 7, _Q), lambda i: (i, 0, 0)),
            pl.BlockSpec((1, 6, _Q), lambda i: (i, 0, 0)),
            pl.BlockSpec((1, 6, _NP), lambda i: (i, 0, 0),
                         memory_space=pltpu.SMEM),
            pl.BlockSpec((6, _Q), lambda i: (0, 0)),
            pl.BlockSpec((1, 1), lambda i: (0, 0), memory_space=pltpu.SMEM),
        ],
        out_specs=pl.BlockSpec((1, 1, 1), lambda i: (i, 0, 0),
                               memory_space=pltpu.SMEM),
        out_shape=jax.ShapeDtypeStruct((_BS, 1, 1), jnp.float32),
    )(predT, sg, gts, aux, coef)
    return (jnp.sum(out) / _BS).reshape(1)


# speedup vs baseline: 1.8813x; 1.8813x over previous
import functools

import jax
import jax.numpy as jnp
import numpy as np
from jax import lax
from jax.experimental import pallas as pl
from jax.experimental.pallas import tpu as pltpu
from jax.experimental.pallas import tpu_sc as plsc

_ANCHORS = [[42.31, 55.41], [102.17, 128.3], [161.79, 259.17], [303.08, 154.9], [359.56, 320.23]]
_ABW = tuple(float(np.float32(np.float32(a[0]) / np.float32(512.0 / 14.0) / np.float32(14.0))) for a in _ANCHORS)
_ABH = tuple(float(np.float32(np.float32(a[1]) / np.float32(512.0 / 14.0) / np.float32(14.0))) for a in _ANCHORS)
_NEG = float(np.float32(-3e38))
_LN2 = 0.6931471805599453
_BS, _A, _N, _NP = 64, 5, 196, 208
_Q = 1024
_CHUNKS = _NP // 16


def _ln(x):
    bits = lax.bitcast_convert_type(x, jnp.int32)
    e = ((bits >> 23) & 0xFF) - 127
    m = lax.bitcast_convert_type((bits & 0x007FFFFF) | 0x3F800000, jnp.float32)
    s = (m - 1.0) / (m + 1.0)
    s2 = s * s
    p = 1.0 + s2 * (1.0 / 3.0 + s2 * (0.2 + s2 * (1.0 / 7.0 + s2 * (1.0 / 9.0))))
    ln = e.astype(jnp.float32) * _LN2 + 2.0 * s * p
    return jnp.where(x > 0.0, ln, -jnp.inf)


def _sig(x):
    return 1.0 / (1.0 + jnp.exp(-x))


def _sc_body(imgs_per, nc, pred_h, lab_h, sg_h, gts_h, pred_v, lab_v, gt_v, sg_v):
    wid = lax.axis_index("s") * nc + lax.axis_index("c")
    lane = lax.iota(jnp.int32, 16)
    zeros16 = jnp.zeros((16,), jnp.float32)

    def one_image(i, carry):
        b = wid * imgs_per + i
        pltpu.sync_copy(pred_h.at[b], pred_v)
        pltpu.sync_copy(lab_h.at[b], lab_v)

        def zbody(k, c):
            sl = pl.ds(k * 16, 16)
            for r in range(6):
                sg_v[r, sl] = zeros16
            return c

        lax.fori_loop(0, _Q // 16, zbody, jnp.int32(0))

        def abody(k, acc):
            acc_t, anyv = acc
            sl = pl.ds(k * 16, 16)
            n_i = lane + k * 16
            colf = lax.rem(n_i, 14).astype(jnp.float32)
            rowf = lax.div(n_i, 14).astype(jnp.float32)
            l0 = lab_v[0, sl]
            l3 = lab_v[3, sl]
            l4 = lab_v[4, sl]
            tw = lab_v[5, sl]
            th = lab_v[6, sl]
            valid = l0 != 0.0
            vldf = jnp.where(valid, 1.0, 0.0).astype(jnp.float32)
            txf = l3 + colf
            tyf = l4 + rowf
            wi = txf.astype(jnp.int32)
            hj = tyf.astype(jnp.int32)
            gtx = txf / 14.0
            gty = tyf / 14.0
            gx1 = gtx - 0.5 * tw
            gx2 = gtx + 0.5 * tw
            gy1 = gty - 0.5 * th
            gy2 = gty + 0.5 * th
            ga = tw * th
            gc = jnp.where(valid, -0.6 * ga, _NEG)
            gt_v[0, sl] = gx1
            gt_v[1, sl] = gx2
            gt_v[2, sl] = gy1
            gt_v[3, sl] = gy2
            gt_v[4, sl] = gc
            best = jnp.full((16,), -1.0, jnp.float32)
            bid = jnp.zeros((16,), jnp.int32)
            aws = jnp.full((16,), _ABW[0], jnp.float32)
            ahs = jnp.full((16,), _ABH[0], jnp.float32)
            for a in range(_A):
                inter = jnp.minimum(tw, _ABW[a]) * jnp.minimum(th, _ABH[a])
                u = ga + (_ABW[a] * _ABH[a]) - inter
                iou = jnp.maximum(inter / u, 0.0)
                cnd = iou > best
                best = jnp.where(cnd, iou, best)
                bid = jnp.where(cnd, a, bid)
                aws = jnp.where(cnd, _ABW[a], aws)
                ahs = jnp.where(cnd, _ABH[a], ahs)
            wg = jnp.minimum(wi, 13)
            hg = jnp.minimum(hj, 13)
            cg = hg * 14 + wg
            ch0 = bid * 7
            p0g = plsc.load_gather(pred_v, [ch0, cg])
            p1g = plsc.load_gather(pred_v, [ch0 + 1, cg])
            p2g = plsc.load_gather(pred_v, [ch0 + 2, cg])
            p3g = plsc.load_gather(pred_v, [ch0 + 3, cg])
            p4g = plsc.load_gather(pred_v, [ch0 + 4, cg])
            p5g = plsc.load_gather(pred_v, [ch0 + 5, cg])
            p6g = plsc.load_gather(pred_v, [ch0 + 6, cg])
            row1 = jnp.full((16,), 1, jnp.int32)
            cv1 = plsc.load_gather(lab_v, [row1, cg])
            cv2 = plsc.load_gather(lab_v, [row1 + 1, cg])
            ax = _sig(p3g)
            ay = _sig(p4g)
            lnw = _ln(tw / aws)
            lnh = _ln(th / ahs)
            truth_x = txf - wi.astype(jnp.float32)
            truth_y = tyf - hj.astype(jnp.float32)
            scale = 2.0 - truth_x * truth_y
            px = (ax + wg.astype(jnp.float32)) / 14.0
            py = (ay + hg.astype(jnp.float32)) / 14.0
            pw = jnp.exp(p5g) * aws
            ph = jnp.exp(p6g) * ahs
            px1 = px - 0.5 * pw
            px2 = px + 0.5 * pw
            py1 = py - 0.5 * ph
            py2 = py + 0.5 * ph
            xi1 = jnp.maximum(px1, gx1)
            xi2 = jnp.minimum(px2, gx2)
            yi1 = jnp.maximum(py1, gy1)
            yi2 = jnp.minimum(py2, gy2)
            inter = jnp.maximum(xi2 - xi1, 0.0) * jnp.maximum(yi2 - yi1, 0.0)
            u = pw * ph + ga - inter
            iou_t = jnp.maximum(inter / u, 0.0)
            obj5 = 5.0 * (p0g - iou_t)
            c1 = scale * (ax - truth_x)
            c2 = scale * (ay - truth_y)
            c3 = scale * (p5g - lnw)
            c4 = scale * (p6g - lnh)
            cls = (p1g - cv1) * (p1g - cv1) + (p2g - cv2) * (p2g - cv2)
            inb = valid & (wi <= 13) & (hj <= 13)
            q = bid * _N + hg * 14 + wg
            onesf = jnp.full((16,), 1.0, jnp.float32)
            for ci, cc in enumerate((c1, c2, c3, c4, obj5, onesf)):
                plsc.addupdate_scatter(sg_v, [jnp.full((16,), ci, jnp.int32), q],
                                       cc, mask=inb)
            acc_t = acc_t + jnp.where(valid, cls, 0.0)
            anyv = jnp.maximum(anyv, vldf)
            return acc_t, anyv

        acc_t, anyv = lax.fori_loop(0, _CHUNKS, abody, (zeros16, zeros16))
        cls_total = jnp.sum(acc_t)
        anym = jnp.max(anyv)
        gt_v[5, pl.ds(0, 16)] = jnp.where(
            lane == 0, cls_total, jnp.where(lane == 1, anym, 0.0))
        pltpu.sync_copy(sg_v, sg_h.at[b])
        pltpu.sync_copy(gt_v, gts_h.at[b])
        return carry

    lax.fori_loop(0, imgs_per, one_image, jnp.int32(0))


def _tc_body(predT_ref, sg_ref, gts_ref, aux_ref, coef_ref, out_ref):
    coefs = coef_ref[0, 0]
    abw = aux_ref[0].reshape(8, 128)
    abh = aux_ref[1].reshape(8, 128)
    colf = aux_ref[2].reshape(8, 128)
    rowf = aux_ref[3].reshape(8, 128)
    p0 = predT_ref[0, 0].reshape(8, 128)
    p3 = predT_ref[0, 3].reshape(8, 128)
    p4 = predT_ref[0, 4].reshape(8, 128)
    p5 = predT_ref[0, 5].reshape(8, 128)
    p6 = predT_ref[0, 6].reshape(8, 128)
    ax = _sig(p3)
    ay = _sig(p4)
    px = (ax + colf) / 14.0
    py = (ay + rowf) / 14.0
    pw = jnp.exp(p5) * abw
    ph = jnp.exp(p6) * abh
    px1 = px - 0.5 * pw
    px2 = px + 0.5 * pw
    py1 = py - 0.5 * ph
    py2 = py + 0.5 * ph
    pam = -0.6 * (pw * ph)

    def gbody(g, m):
        gx1s = gts_ref[0, 0, g]
        gx2s = gts_ref[0, 1, g]
        gy1s = gts_ref[0, 2, g]
        gy2s = gts_ref[0, 3, g]
        gcs = gts_ref[0, 4, g]
        xi1 = jnp.maximum(px1, gx1s)
        xi2 = jnp.minimum(px2, gx2s)
        yi1 = jnp.maximum(py1, gy1s)
        yi2 = jnp.minimum(py2, gy2s)
        inter = jnp.maximum(xi2 - xi1, 0.0) * jnp.maximum(yi2 - yi1, 0.0)
        return jnp.maximum(m, 1.6 * inter + gcs)

    m = lax.fori_loop(0, _N, gbody, jnp.full((8, 128), _NEG, jnp.float32),
                      unroll=14)
    noobj = (m + pam) <= 0.0
    ol = jnp.where(noobj, 0.5 * p0, 0.0)
    mc = sg_ref[0, 5].reshape(8, 128)
    omc = 1.0 - mc
    acc = jnp.zeros((8, 128), jnp.float32)
    for ci, co in enumerate((coefs * (ax - 0.5), coefs * (ay - 0.5),
                             coefs * p5, coefs * p6, ol)):
        tg = sg_ref[0, ci].reshape(8, 128)
        bc = mc * tg + omc * co
        acc = acc + bc * bc
    cls_s = gts_ref[0, 5, 0]
    anyv_s = gts_ref[0, 5, 1]
    pb = jnp.where(anyv_s > 0.0, jnp.sum(acc) + cls_s, 0.0)
    out_ref[0, 0, 0] = pb


def kernel(pred, label, seen):
    try:
        info = plsc.get_sparse_core_info()
        nc, ns = info.num_cores, info.num_subcores
    except Exception:
        nc, ns = 2, 16
    nw = nc * ns
    imgs_per = _BS // nw
    pred2 = jnp.pad(pred.reshape(_BS, 35, _N), ((0, 0), (0, 0), (0, _NP - _N)))
    lab2 = jnp.pad(label.reshape(_BS, 7, _N), ((0, 0), (0, 0), (0, _NP - _N)))
    predT = jnp.pad(
        pred.reshape(_BS, _A, 7, _N).transpose(0, 2, 1, 3).reshape(_BS, 7, _A * _N),
        ((0, 0), (0, 0), (0, _Q - _A * _N)))
    coef = jnp.where(jnp.asarray(seen) < 12800, jnp.float32(0.01),
                     jnp.float32(0.0)).reshape(1, 1)
    q = np.arange(_Q)
    a_of_q = np.minimum(q // _N, _A - 1)
    n_of_q = q % _N
    aux = np.zeros((6, _Q), np.float32)
    aux[0] = np.where(q < _A * _N, np.asarray(_ABW, np.float32)[a_of_q], 1.0)
    aux[1] = np.where(q < _A * _N, np.asarray(_ABH, np.float32)[a_of_q], 1.0)
    aux[2] = np.where(q < _A * _N, (n_of_q % 14).astype(np.float32), 0.0)
    aux[3] = np.where(q < _A * _N, (n_of_q // 14).astype(np.float32), 0.0)
    aux = jnp.asarray(aux)

    mesh = plsc.VectorSubcoreMesh(core_axis_name="c", subcore_axis_name="s",
                                  num_cores=nc, num_subcores=ns)
    sg, gts = pl.kernel(
        functools.partial(_sc_body, imgs_per, nc),
        out_type=(jax.ShapeDtypeStruct((_BS, 6, _Q), jnp.float32),
                  jax.ShapeDtypeStruct((_BS, 6, _NP), jnp.float32)),
        mesh=mesh,
        compiler_params=pltpu.CompilerParams(use_tc_tiling_on_sc=False,
                                             needs_layout_passes=False),
        scratch_types=[
            pltpu.VMEM((35, _NP), jnp.float32),
            pltpu.VMEM((7, _NP), jnp.float32),
            pltpu.VMEM((6, _NP), jnp.float32),
            pltpu.VMEM((6, _Q), jnp.float32),
        ],
    )(pred2, lab2)

    out = pl.pallas_call(
        _tc_body,
        grid=(_BS,),
        in_specs=[
            pl.BlockSpec((1, 7, _Q), lambda i: (i, 0, 0)),
            pl.BlockSpec((1, 6, _Q), lambda i: (i, 0, 0)),
            pl.BlockSpec((1, 6, _NP), lambda i: (i, 0, 0),
                         memory_space=pltpu.SMEM),
            pl.BlockSpec((6, _Q), lambda i: (0, 0)),
            pl.BlockSpec((1, 1), lambda i: (0, 0), memory_space=pltpu.SMEM),
        ],
        out_specs=pl.BlockSpec((1, 1, 1), lambda i: (i, 0, 0),
                               memory_space=pltpu.SMEM),
        out_shape=jax.ShapeDtypeStruct((_BS, 1, 1), jnp.float32),
    )(predT, sg, gts, aux, coef)
    return (jnp.sum(out) / _BS).reshape(1)
